# baseline (speedup 1.0000x reference)
import functools

import jax
import jax.numpy as jnp
from jax import lax
from jax.experimental import pallas as pl
from jax.experimental.pallas import tpu as pltpu
from jax.experimental.pallas import tpu_sc as plsc

NUM_TOK = 16384 * 50
DIM = 64
NC = 2
NS = 16
NW = NC * NS
PER_W = NUM_TOK // NW
HALF = PER_W // 2
C = 512
CH2 = C // 2
NCH = PER_W // C

_mesh = plsc.VectorSubcoreMesh(core_axis_name="c", subcore_axis_name="s")


@functools.partial(
    pl.kernel,
    mesh=_mesh,
    out_type=jax.ShapeDtypeStruct((NUM_TOK // 2, 2 * DIM), jnp.float32),
    scratch_types=[
        pltpu.VMEM((PER_W,), jnp.int32),
        pltpu.VMEM((CH2, DIM), jnp.float32),
        pltpu.VMEM((CH2, DIM), jnp.float32),
        pltpu.VMEM((CH2, DIM), jnp.float32),
        pltpu.VMEM((CH2, DIM), jnp.float32),
        pltpu.SemaphoreType.DMA,
        pltpu.SemaphoreType.DMA,
        pltpu.SemaphoreType.DMA,
        pltpu.SemaphoreType.DMA,
    ],
    compiler_params=pltpu.CompilerParams(use_tc_tiling_on_sc=False),
)
def _gather(idx_hbm, table_hbm, out_hbm, idx_v,
            rowsE0, rowsE1, rowsO0, rowsO1,
            gsem0, gsem1, ssem0, ssem1):
    wid = lax.axis_index("s") * NC + lax.axis_index("c")
    ibase = wid * PER_W
    obase = wid * HALF
    rowsE = (rowsE0, rowsE1)
    rowsO = (rowsO0, rowsO1)
    gsem = (gsem0, gsem1)
    ssem = (ssem0, ssem1)

    pltpu.sync_copy(idx_hbm.at[pl.ds(ibase, PER_W)], idx_v)

    def fire_gathers(c, b):
        pltpu.async_copy(
            table_hbm.at[idx_v.at[pl.ds(c * CH2, CH2)]],
            rowsE[b], gsem[b])
        pltpu.async_copy(
            table_hbm.at[idx_v.at[pl.ds(HALF + c * CH2, CH2)]],
            rowsO[b], gsem[b])

    def drain(sem, b):
        pltpu.make_async_copy(
            table_hbm.at[pl.ds(0, CH2)], rowsE[b], sem[b]).wait()
        pltpu.make_async_copy(
            table_hbm.at[pl.ds(0, CH2)], rowsO[b], sem[b]).wait()

    def fire_stores(c, b):
        r0 = obase + c * CH2
        pltpu.async_copy(
            rowsE[b], out_hbm.at[pl.ds(r0, CH2), pl.ds(0, DIM)], ssem[b])
        pltpu.async_copy(
            rowsO[b], out_hbm.at[pl.ds(r0, CH2), pl.ds(DIM, DIM)], ssem[b])

    def body(g, carry):
        for b in (0, 1):
            c = 2 * g + b
            @pl.when(g >= 1)
            def _():
                drain(ssem, b)
            fire_gathers(c, b)
        for b in (0, 1):
            c = 2 * g + b
            drain(gsem, b)
            fire_stores(c, b)
        return carry

    lax.fori_loop(0, NCH // 2, body, 0)
    drain(ssem, 0)
    drain(ssem, 1)


def kernel(tokens_ids, embedding_tensor):
    flat = tokens_ids.reshape(-1).astype(jnp.int32)
    idxp = flat.reshape(NW, HALF, 2).transpose(0, 2, 1).reshape(-1)
    t128 = lax.optimization_barrier(embedding_tensor.reshape(500000, 2 * DIM))
    t64 = t128.reshape(1000000, DIM)
    out = _gather(idxp, t64)
    return out.reshape(16384, 50, DIM)


# speedup vs baseline: 1.1136x; 1.1136x over previous
import functools

import jax
import jax.numpy as jnp
from jax import lax
from jax.experimental import pallas as pl
from jax.experimental.pallas import tpu as pltpu
from jax.experimental.pallas import tpu_sc as plsc

B = 16384
S = 50
DIM = 64
NUM_TOK = B * S
NC = 2
NS = 16
NW = NC * NS
PER_W = NUM_TOK // NW
BPW = B // NW
C = 400
CB = C // S
NCH = PER_W // C

_mesh = plsc.VectorSubcoreMesh(core_axis_name="c", subcore_axis_name="s")


@functools.partial(
    pl.kernel,
    mesh=_mesh,
    out_type=jax.ShapeDtypeStruct((B, S, DIM), jnp.float32),
    scratch_types=[
        pltpu.VMEM((PER_W,), jnp.int32),
        pltpu.VMEM((C, DIM), jnp.float32),
        pltpu.VMEM((C, DIM), jnp.float32),
        pltpu.SemaphoreType.DMA,
        pltpu.SemaphoreType.DMA,
        pltpu.SemaphoreType.DMA,
        pltpu.SemaphoreType.DMA,
    ],
    compiler_params=pltpu.CompilerParams(use_tc_tiling_on_sc=False),
)
def _gather(idx_hbm, table_hbm, out_hbm, idx_v, rows0, rows1,
            gsem0, gsem1, ssem0, ssem1):
    wid = lax.axis_index("s") * NC + lax.axis_index("c")
    ibase = wid * PER_W
    bbase = wid * BPW
    rows = (rows0, rows1)
    gsem = (gsem0, gsem1)
    ssem = (ssem0, ssem1)

    pltpu.sync_copy(idx_hbm.at[pl.ds(ibase, PER_W)], idx_v)

    def fire_gather(c, b):
        pltpu.async_copy(
            table_hbm.at[idx_v.at[pl.ds(c * C, C)]],
            rows[b], gsem[b])

    def drain_gather(b):
        pltpu.make_async_copy(
            table_hbm.at[pl.ds(0, C)], rows[b], gsem[b]).wait()

    def fire_stores(c, b):
        for k in range(CB):
            pltpu.async_copy(
                rows[b].at[pl.ds(k * S, S)],
                out_hbm.at[bbase + c * CB + k],
                ssem[b])

    def drain_stores(b):
        for k in range(CB):
            pltpu.make_async_copy(
                table_hbm.at[pl.ds(0, S)],
                rows[b].at[pl.ds(k * S, S)],
                ssem[b]).wait()

    def body(g, carry):
        for b in (0, 1):
            c = 2 * g + b
            @pl.when(g >= 1)
            def _():
                drain_stores(b)
            fire_gather(c, b)
        for b in (0, 1):
            c = 2 * g + b
            drain_gather(b)
            fire_stores(c, b)
        return carry

    lax.fori_loop(0, NCH // 2, body, 0)
    drain_stores(0)
    drain_stores(1)


def kernel(tokens_ids, embedding_tensor):
    flat = tokens_ids.reshape(-1).astype(jnp.int32)
    t64 = lax.optimization_barrier(embedding_tensor.T).T
    return _gather(flat, t64)
